# baseline (speedup 1.0000x reference)
import dataclasses
import functools

import jax
import jax.numpy as jnp
from jax import lax
from jax.experimental import pallas as pl
from jax.experimental.pallas import tpu as pltpu
from jax.experimental.pallas import tpu_sc as plsc

NUM_EXPERTS = 64
EMBED_DIM = 2048
N_TOKENS = 16384
TOP_K = 8

_NC, _NS = 2, 16
_NW = _NC * _NS
_B = N_TOKENS * TOP_K
_SEG = 4096
_NSEG = _B // _SEG
_VPS = _SEG // 16
_CAP = _SEG + 64
_W = 24
_NSLOT = 8


def _sc_scatter(idx_flat, table):
    mesh = plsc.VectorSubcoreMesh(core_axis_name="c", subcore_axis_name="s")
    cp = pltpu.CompilerParams()
    if "needs_layout_passes" in pltpu.CompilerParams.__dataclass_fields__:
        cp = dataclasses.replace(cp, needs_layout_passes=False)

    @functools.partial(
        pl.kernel,
        out_type=jax.ShapeDtypeStruct((_B, EMBED_DIM), jnp.float32),
        mesh=mesh,
        compiler_params=cp,
        scratch_types=[
            pltpu.VMEM((_SEG,), jnp.int32),
            pltpu.VMEM((_SEG,), jnp.int32),
            pltpu.VMEM((_CAP,), jnp.int32),
            pltpu.VMEM((_CAP,), jnp.int32),
            pltpu.VMEM((_W, EMBED_DIM), jnp.float32),
            pltpu.VMEM((_W, EMBED_DIM), jnp.float32),
            pltpu.VMEM((_NSLOT, _W), jnp.int32),
            pltpu.SMEM((16,), jnp.int32),
            pltpu.SemaphoreType.DMA,
            pltpu.SemaphoreType.DMA,
            pltpu.SemaphoreType.DMA,
            pltpu.SemaphoreType.DMA,
            pltpu.SemaphoreType.DMA,
            pltpu.SemaphoreType.DMA,
            pltpu.SemaphoreType.DMA,
            pltpu.SemaphoreType.DMA,
            pltpu.SemaphoreType.DMA,
        ],
    )
    def k(table_hbm, idx_hbm, out_hbm, segA, segB, pos0, pos1, rep0, rep1,
          stg, cnts, gsem, ws0, ws1, ws2, ws3, ws4, ws5, ws6, ws7):
        wsems = (ws0, ws1, ws2, ws3, ws4, ws5, ws6, ws7)
        wid = lax.axis_index("s") * _NC + lax.axis_index("c")
        e0 = wid * 2
        lanes = lax.iota(jnp.int32, 16)
        for j in range(_NSLOT):
            cnts[4 + j] = 0
        cnts[0] = 0
        cnts[1] = 0
        cnts[2] = 0

        def slot_wait(j, rep):
            @pl.when(cnts[4 + j] > 0)
            def _():
                pltpu.make_async_copy(rep, out_hbm.at[stg.at[j]],
                                      wsems[j]).wait()

        for sl, rep in ((0, rep0), (1, rep1)):
            pltpu.sync_copy(table_hbm.at[pl.ds(e0 + sl, 1)],
                            rep.at[pl.ds(0, 1)])

        @pl.loop(0, EMBED_DIM // 16)
        def _(jc):
            col = pl.ds(jc * 16, 16)
            v0 = rep0[0, col]
            v1 = rep1[0, col]
            for w in range(1, _W):
                rep0[w, col] = v0
                rep1[w, col] = v1

        pltpu.async_copy(idx_hbm.at[pl.ds(0, _SEG)], segA, gsem)

        def do_segment(seg, cur, nxt):
            pltpu.make_async_copy(idx_hbm.at[pl.ds(0, _SEG)], cur,
                                  gsem).wait()

            @pl.when(seg + 1 < _NSEG)
            def _():
                pltpu.async_copy(
                    idx_hbm.at[pl.ds((seg + 1) * _SEG, _SEG)], nxt, gsem)

            @pl.loop(0, _VPS, unroll=8,
                     init_carry=(cnts[0], cnts[1]))
            def scan(i, carry):
                cnt0, cnt1 = carry
                v = cur[pl.ds(i * 16, 16)]
                ps = (seg * _SEG + i * 16) + lanes
                m0 = v == e0
                m1 = v == (e0 + 1)
                plsc.store_compressed(pos0.at[pl.ds(cnt0, 16)], ps,
                                      mask=m0)
                plsc.store_compressed(pos1.at[pl.ds(cnt1, 16)], ps,
                                      mask=m1)
                c0 = jnp.max(plsc.all_reduce_population_count(m0))
                c1 = jnp.max(plsc.all_reduce_population_count(m1))
                return (cnt0 + c0, cnt1 + c1)

            cnt0, cnt1 = scan

            for sl, pref, rep, cnt in ((0, pos0, rep0, cnt0),
                                       (1, pos1, rep1, cnt1)):
                nb = cnt // _W

                @pl.loop(0, nb)
                def _(kk):
                    j = (cnts[2] + kk) % _NSLOT
                    for jj in range(_NSLOT):
                        @pl.when(j == jj)
                        def _():
                            slot_wait(jj, rep)
                            stg[jj, pl.ds(0, 16)] = pref[pl.ds(kk * _W,
                                                               16)]
                            stg[jj, pl.ds(8, 16)] = (
                                pref[pl.ds(kk * _W + 8, 16)])
                            pltpu.async_copy(rep, out_hbm.at[stg.at[jj]],
                                             wsems[jj])
                            cnts[4 + jj] = 1

                cnts[2] = (cnts[2] + nb) % _NSLOT

                @pl.when(nb > 0)
                def _():
                    lo = pref[pl.ds(nb * _W, 16)]
                    hi = pref[pl.ds(nb * _W + 8, 16)]
                    pref[pl.ds(0, 16)] = lo
                    pref[pl.ds(8, 16)] = hi
                cnts[sl] = cnt - nb * _W

        @pl.loop(0, _NSEG, step=2)
        def _(seg):
            do_segment(seg, segA, segB)
            do_segment(seg + 1, segB, segA)

        for sl, pref, rep, jj in ((0, pos0, rep0, 0), (1, pos1, rep1, 1)):
            cnt = cnts[sl]

            @pl.when(cnt > 0)
            def _():
                slot_wait(jj, rep)
                last = plsc.load_gather(
                    pref, [jnp.full((16,), cnt - 1, jnp.int32)])
                c0 = pref[pl.ds(0, 16)]
                c1 = pref[pl.ds(8, 16)]
                stg[jj, pl.ds(0, 16)] = jnp.where(lanes < cnt, c0, last)
                stg[jj, pl.ds(8, 16)] = jnp.where(lanes + 8 < cnt, c1,
                                                  last)
                pltpu.async_copy(rep, out_hbm.at[stg.at[jj]], wsems[jj])
                cnts[4 + jj] = 1

        for jj in range(_NSLOT):
            @pl.when(cnts[4 + jj] > 0)
            def _():
                pltpu.make_async_copy(rep0, out_hbm.at[stg.at[jj]],
                                      wsems[jj]).wait()

    return k(table, idx_flat)


def kernel(expert_indices, expert_embeddings_weight):
    idx = expert_indices.reshape(-1).astype(jnp.int32)
    out = _sc_scatter(idx, expert_embeddings_weight)
    return out.reshape(N_TOKENS, TOP_K, EMBED_DIM)


# speedup vs baseline: 1.0011x; 1.0011x over previous
import dataclasses
import functools

import jax
import jax.numpy as jnp
from jax import lax
from jax.experimental import pallas as pl
from jax.experimental.pallas import tpu as pltpu
from jax.experimental.pallas import tpu_sc as plsc

NUM_EXPERTS = 64
EMBED_DIM = 2048
N_TOKENS = 16384
TOP_K = 8

_NC, _NS = 2, 16
_NW = _NC * _NS
_B = N_TOKENS * TOP_K
_SEG = 4096
_NSEG = _B // _SEG
_VPS = _SEG // 16
_CAP = _SEG + 64
_W = 24
_NSLOT = 8


def _sc_scatter(idx_flat, table):
    mesh = plsc.VectorSubcoreMesh(core_axis_name="c", subcore_axis_name="s")
    cp = pltpu.CompilerParams()
    if "needs_layout_passes" in pltpu.CompilerParams.__dataclass_fields__:
        cp = dataclasses.replace(cp, needs_layout_passes=False)

    @functools.partial(
        pl.kernel,
        out_type=jax.ShapeDtypeStruct((_B, EMBED_DIM), jnp.float32),
        mesh=mesh,
        compiler_params=cp,
        scratch_types=[
            pltpu.VMEM((_SEG,), jnp.int32),
            pltpu.VMEM((_SEG,), jnp.int32),
            pltpu.VMEM((_CAP,), jnp.int32),
            pltpu.VMEM((_CAP,), jnp.int32),
            pltpu.VMEM((_W, EMBED_DIM), jnp.float32),
            pltpu.VMEM((_W, EMBED_DIM), jnp.float32),
            pltpu.VMEM((_NSLOT, _W), jnp.int32),
            pltpu.SMEM((16,), jnp.int32),
            pltpu.SemaphoreType.DMA,
            pltpu.SemaphoreType.DMA,
            pltpu.SemaphoreType.DMA,
            pltpu.SemaphoreType.DMA,
            pltpu.SemaphoreType.DMA,
            pltpu.SemaphoreType.DMA,
            pltpu.SemaphoreType.DMA,
            pltpu.SemaphoreType.DMA,
            pltpu.SemaphoreType.DMA,
        ],
    )
    def k(table_hbm, idx_hbm, out_hbm, segA, segB, pos0, pos1, rep0, rep1,
          stg, cnts, gsem, ws0, ws1, ws2, ws3, ws4, ws5, ws6, ws7):
        wsems = (ws0, ws1, ws2, ws3, ws4, ws5, ws6, ws7)
        wid = lax.axis_index("s") * _NC + lax.axis_index("c")
        e0 = wid * 2
        lanes = lax.iota(jnp.int32, 16)
        for j in range(_NSLOT):
            cnts[4 + j] = 0
        cnts[0] = 0
        cnts[1] = 0
        cnts[2] = 0

        def slot_wait(j, rep):
            @pl.when(cnts[4 + j] > 0)
            def _():
                pltpu.make_async_copy(rep, out_hbm.at[stg.at[j]],
                                      wsems[j]).wait()

        for sl, rep in ((0, rep0), (1, rep1)):
            pltpu.sync_copy(table_hbm.at[pl.ds(e0 + sl, 1)],
                            rep.at[pl.ds(0, 1)])

        @pl.loop(0, EMBED_DIM // 16)
        def _(jc):
            col = pl.ds(jc * 16, 16)
            v0 = rep0[0, col]
            v1 = rep1[0, col]
            for w in range(1, _W):
                rep0[w, col] = v0
                rep1[w, col] = v1

        pltpu.async_copy(idx_hbm.at[pl.ds(0, _SEG)], segA, gsem)

        def do_segment(seg, cur, nxt):
            pltpu.make_async_copy(idx_hbm.at[pl.ds(0, _SEG)], cur,
                                  gsem).wait()

            @pl.when(seg + 1 < _NSEG)
            def _():
                pltpu.async_copy(
                    idx_hbm.at[pl.ds((seg + 1) * _SEG, _SEG)], nxt, gsem)

            @pl.loop(0, _VPS, unroll=8,
                     init_carry=(jnp.full((16,), cnts[0], jnp.int32),
                                 jnp.full((16,), cnts[1], jnp.int32)))
            def scan(i, carry):
                c0v, c1v = carry
                v = cur[pl.ds(i * 16, 16)]
                ps = (seg * _SEG + i * 16) + lanes
                m0 = v == e0
                m1 = v == (e0 + 1)
                r0 = plsc.cumsum(m0.astype(jnp.int32))
                r1 = plsc.cumsum(m1.astype(jnp.int32))
                plsc.store_scatter(pos0, [c0v + r0 - 1], ps, mask=m0)
                plsc.store_scatter(pos1, [c1v + r1 - 1], ps, mask=m1)
                return (c0v + plsc.all_reduce_population_count(m0),
                        c1v + plsc.all_reduce_population_count(m1))

            c0v, c1v = scan
            cnt0 = jnp.max(c0v)
            cnt1 = jnp.max(c1v)

            for sl, pref, rep, cnt in ((0, pos0, rep0, cnt0),
                                       (1, pos1, rep1, cnt1)):
                nb = cnt // _W

                @pl.loop(0, nb)
                def _(kk):
                    j = (cnts[2] + kk) % _NSLOT
                    for jj in range(_NSLOT):
                        @pl.when(j == jj)
                        def _():
                            slot_wait(jj, rep)
                            stg[jj, pl.ds(0, 16)] = pref[pl.ds(kk * _W,
                                                               16)]
                            stg[jj, pl.ds(8, 16)] = (
                                pref[pl.ds(kk * _W + 8, 16)])
                            pltpu.async_copy(rep, out_hbm.at[stg.at[jj]],
                                             wsems[jj])
                            cnts[4 + jj] = 1

                cnts[2] = (cnts[2] + nb) % _NSLOT

                @pl.when(nb > 0)
                def _():
                    lo = pref[pl.ds(nb * _W, 16)]
                    hi = pref[pl.ds(nb * _W + 8, 16)]
                    pref[pl.ds(0, 16)] = lo
                    pref[pl.ds(8, 16)] = hi
                cnts[sl] = cnt - nb * _W

        @pl.loop(0, _NSEG, step=2)
        def _(seg):
            do_segment(seg, segA, segB)
            do_segment(seg + 1, segB, segA)

        for sl, pref, rep, jj in ((0, pos0, rep0, 0), (1, pos1, rep1, 1)):
            cnt = cnts[sl]

            @pl.when(cnt > 0)
            def _():
                slot_wait(jj, rep)
                last = plsc.load_gather(
                    pref, [jnp.full((16,), cnt - 1, jnp.int32)])
                c0 = pref[pl.ds(0, 16)]
                c1 = pref[pl.ds(8, 16)]
                stg[jj, pl.ds(0, 16)] = jnp.where(lanes < cnt, c0, last)
                stg[jj, pl.ds(8, 16)] = jnp.where(lanes + 8 < cnt, c1,
                                                  last)
                pltpu.async_copy(rep, out_hbm.at[stg.at[jj]], wsems[jj])
                cnts[4 + jj] = 1

        for jj in range(_NSLOT):
            @pl.when(cnts[4 + jj] > 0)
            def _():
                pltpu.make_async_copy(rep0, out_hbm.at[stg.at[jj]],
                                      wsems[jj]).wait()

    return k(table, idx_flat)


def kernel(expert_indices, expert_embeddings_weight):
    idx = expert_indices.reshape(-1).astype(jnp.int32)
    out = _sc_scatter(idx, expert_embeddings_weight)
    return out.reshape(N_TOKENS, TOP_K, EMBED_DIM)
